# baseline (speedup 1.0000x reference)
import functools

import jax
import jax.numpy as jnp
from jax import lax
from jax.experimental import pallas as pl
from jax.experimental.pallas import tpu as pltpu
from jax.experimental.pallas import tpu_sc as plsc

N = 10000
E = 320000
D_FEAT = 128
H1 = 16
H2 = 32

NC = 2
NS = 16
L = 16
NW = NC * NS

BLK = 128
BLKS_PER_TILE = 80
MAIN_BLKS = E // BLK
MAIN_LAST_BASE = (NW - 1) * BLKS_PER_TILE
LASTW_MAIN = MAIN_BLKS - MAIN_LAST_BASE
PAD_BLKS = NW * BLKS_PER_TILE - MAIN_BLKS
NPAD = 10112
RPT = NPAD // NS

@functools.cache
def _mesh():
    return plsc.VectorSubcoreMesh(core_axis_name="c", subcore_axis_name="s",
                                  num_cores=NC, num_subcores=NS)


def _fill_zeros(ref, nrows, ncols):
    @pl.loop(0, nrows)
    def _z(r):
        for hh in range(ncols // L):
            ref[r, pl.ds(hh * L, L)] = jnp.zeros((L,), jnp.float32)


NBUF = 8


def _load_indices(main_ref, pad_ref, vref, w):
    @pl.when(w < NW - 1)
    def _full(_=None):
        pltpu.sync_copy(main_ref.at[pl.ds(w * BLKS_PER_TILE, BLKS_PER_TILE)],
                        vref)

    @pl.when(w == NW - 1)
    def _mixed(_=None):
        pltpu.sync_copy(main_ref.at[pl.ds(MAIN_LAST_BASE, LASTW_MAIN)],
                        vref.at[pl.ds(0, LASTW_MAIN)])
        pltpu.sync_copy(pad_ref, vref.at[pl.ds(LASTW_MAIN, PAD_BLKS)])


@functools.cache
def _make_agg(H):
    @functools.partial(
        pl.kernel,
        out_type=jax.ShapeDtypeStruct((NC, NPAD, H), jnp.float32),
        mesh=_mesh(),
        compiler_params=pltpu.CompilerParams(use_tc_tiling_on_sc=False),
        scratch_types=[
            pltpu.VMEM((BLKS_PER_TILE, BLK), jnp.int32),
            pltpu.VMEM((BLKS_PER_TILE, BLK), jnp.int32),
            pltpu.VMEM((RPT, H), jnp.float32),
            pltpu.VMEM_SHARED((NPAD, H), jnp.float32),
        ]
        + [pltpu.VMEM((BLK, H), jnp.float32) for _ in range(NBUF)]
        + [pltpu.SemaphoreType.DMA for _ in range(2 * NBUF)],
    )
    def agg(hs, msrc, mdst, psrc, pdst, out, src_v, dst_v, zer_v, acc_sh, *rb):
        rows = rb[:NBUF]
        gsem = rb[NBUF:2 * NBUF]
        ssem = rb[2 * NBUF:]
        cid = lax.axis_index("c")
        sid = lax.axis_index("s")
        w = cid * NS + sid

        _fill_zeros(zer_v, RPT, H)
        pltpu.sync_copy(zer_v, acc_sh.at[pl.ds(sid * RPT, RPT)])
        plsc.subcore_barrier()

        _load_indices(msrc, psrc, src_v, w)
        _load_indices(mdst, pdst, dst_v, w)

        for b in range(NBUF):
            pltpu.async_copy(hs.at[src_v.at[b]], rows[b], gsem[b])

        @pl.loop(0, BLKS_PER_TILE, step=NBUF)
        def _chunk(j):
            for b in range(NBUF):
                pltpu.make_async_copy(hs.at[pl.ds(0, BLK)], rows[b],
                                      gsem[b]).wait()
                pltpu.async_copy(rows[b], acc_sh.at[dst_v.at[j + b]],
                                 ssem[b], add=True)
            for b in range(NBUF):
                pltpu.make_async_copy(rows[b], acc_sh.at[dst_v.at[j]],
                                      ssem[b]).wait()

                @pl.when(j + b + NBUF < BLKS_PER_TILE)
                def _refill(b=b):
                    pltpu.async_copy(hs.at[src_v.at[j + b + NBUF]],
                                     rows[b], gsem[b])

        plsc.subcore_barrier()
        pltpu.sync_copy(acc_sh.at[pl.ds(sid * RPT, RPT)],
                        out.at[cid, pl.ds(sid * RPT, RPT)])

    return agg


DEGL = 16

F16 = N * H1 // 128
F16P = NPAD * H1 // 128
F32 = N * H2 // 128
F32P = NPAD * H2 // 128


@functools.cache
def _make_deg():
    @functools.partial(
        pl.kernel,
        out_type=jax.ShapeDtypeStruct((NC, NPAD, DEGL), jnp.float32),
        mesh=_mesh(),
        compiler_params=pltpu.CompilerParams(use_tc_tiling_on_sc=False),
        scratch_types=[
            pltpu.VMEM((BLKS_PER_TILE, BLK), jnp.int32),
            pltpu.VMEM((BLK, DEGL), jnp.float32),
            pltpu.VMEM_SHARED((NPAD, DEGL), jnp.float32),
            pltpu.SemaphoreType.DMA,
        ],
    )
    def _deg(mdst, pdst, ones8, zer8, out, dst_v, ones_v, acc_sh, dsem):
        cid = lax.axis_index("c")
        sid = lax.axis_index("s")
        w = cid * NS + sid

        pltpu.sync_copy(ones8, ones_v)
        pltpu.sync_copy(zer8, acc_sh.at[pl.ds(sid * RPT, RPT)])
        plsc.subcore_barrier()

        _load_indices(mdst, pdst, dst_v, w)

        @pl.loop(0, BLKS_PER_TILE)
        def _edge_block(j):
            pltpu.async_copy(ones_v, acc_sh.at[dst_v.at[j]], dsem, add=True)

        @pl.loop(0, BLKS_PER_TILE)
        def _drain(j):
            pltpu.make_async_copy(ones_v, acc_sh.at[dst_v.at[0]], dsem).wait()

        plsc.subcore_barrier()
        pltpu.sync_copy(acc_sh.at[pl.ds(sid * RPT, RPT)],
                        out.at[cid, pl.ds(sid * RPT, RPT)])

    return _deg


def _tc01_body(degf_ref, xf_ref, bdw1_ref, h1sf_ref, ds16_ref):
    d = degf_ref[0, :F16, :] + degf_ref[1, :F16, :] + 1.0
    ds = lax.rsqrt(d)
    h1 = jnp.dot(xf_ref[...], bdw1_ref[...],
                 preferred_element_type=jnp.float32)
    h1sf_ref[...] = h1 * ds
    ds16_ref[...] = ds


def _tc2_body(p_ref, h1sf_ref, ds16_ref, b1f_ref, bdw2_ref, h2sf_ref):
    agg = p_ref[0, :F16, :] + p_ref[1, :F16, :] + h1sf_ref[...]
    z1 = jnp.maximum(agg * ds16_ref[...] + b1f_ref[...], 0.0)
    h2sf_ref[...] = jnp.dot(z1 * ds16_ref[...], bdw2_ref[...],
                            preferred_element_type=jnp.float32)


def _tc3_body(q_ref, h2sf_ref, ds32_ref, b2f_ref, wfcf_ref, bfc_ref, out_ref):
    agg = q_ref[0, :F32, :] + q_ref[1, :F32, :] + h2sf_ref[...]
    z2 = jnp.maximum(agg * ds32_ref[...] + b2f_ref[...], 0.0)
    out_ref[...] = jnp.sum(z2 * wfcf_ref[...]) * (1.0 / N) + bfc_ref[...]


def kernel(x, edge_index, W1, b1, W2, b2, Wfc, bfc):
    ei = edge_index.astype(jnp.int32)
    msrc = ei[0].reshape(MAIN_BLKS, BLK)
    mdst = ei[1].reshape(MAIN_BLKS, BLK)
    ar = jnp.arange(PAD_BLKS * BLK, dtype=jnp.int32)
    psrc = ((ar * 97) % N).reshape(PAD_BLKS, BLK)
    pdst = (N + ar % (NPAD - N)).reshape(PAD_BLKS, BLK)

    ones16 = jnp.ones((BLK, DEGL), jnp.float32)
    zer16 = jnp.zeros((RPT, DEGL), jnp.float32)
    degp = _make_deg()(mdst, pdst, ones16, zer16)
    degf = degp.reshape(NC, F16P, 128)

    eye8 = jnp.eye(8, dtype=jnp.float32)
    bdw1 = jnp.kron(eye8, W1)
    bdw2 = jnp.kron(eye8, W2)
    b1f = jnp.tile(b1, 8)
    b2f = jnp.tile(b2, 4)
    wfcf = jnp.tile(Wfc[:, 0], 4)
    xf = x.reshape(N * D_FEAT // 1024, 1024)

    h1sf, ds16 = pl.pallas_call(
        _tc01_body,
        out_shape=[
            jax.ShapeDtypeStruct((F16, 128), jnp.float32),
            jax.ShapeDtypeStruct((F16, 128), jnp.float32),
        ],
    )(degf, xf, bdw1)

    p1 = _make_agg(H1)(h1sf.reshape(N, H1), msrc, mdst, psrc, pdst)

    h2sf256 = pl.pallas_call(
        _tc2_body,
        out_shape=jax.ShapeDtypeStruct((F16, 256), jnp.float32),
    )(p1.reshape(NC, F16P, 128), h1sf, ds16, b1f, bdw2)
    h2sf = h2sf256.reshape(F32, 128)
    ds32 = jnp.repeat(ds16, 2, axis=1).reshape(F32, 128)

    p2 = _make_agg(H2)(h2sf.reshape(N, H2), msrc, mdst, psrc, pdst)

    out = pl.pallas_call(
        _tc3_body,
        out_shape=jax.ShapeDtypeStruct((1,), jnp.float32),
    )(p2.reshape(NC, F32P, 128), h2sf, ds32, b2f, wfcf, bfc)

    return out


# speedup vs baseline: 1.5771x; 1.5771x over previous
import functools

import jax
import jax.numpy as jnp
from jax import lax
from jax.experimental import pallas as pl
from jax.experimental.pallas import tpu as pltpu
from jax.experimental.pallas import tpu_sc as plsc

N = 10000
E = 320000
D_FEAT = 128
H1 = 16
H2 = 32

NC = 2
NS = 16
L = 16
NW = NC * NS

BLK = 128
BLKS_PER_TILE = 80
MAIN_BLKS = E // BLK
MAIN_LAST_BASE = (NW - 1) * BLKS_PER_TILE
LASTW_MAIN = MAIN_BLKS - MAIN_LAST_BASE
PAD_BLKS = NW * BLKS_PER_TILE - MAIN_BLKS
NPAD = 10112
RPT = NPAD // NS

@functools.cache
def _mesh():
    return plsc.VectorSubcoreMesh(core_axis_name="c", subcore_axis_name="s",
                                  num_cores=NC, num_subcores=NS)


def _fill_zeros(ref, nrows, ncols):
    @pl.loop(0, nrows)
    def _z(r):
        for hh in range(ncols // L):
            ref[r, pl.ds(hh * L, L)] = jnp.zeros((L,), jnp.float32)


NBUF = 8


def _load_indices(main_ref, pad_ref, vref, w):
    @pl.when(w < NW - 1)
    def _full(_=None):
        pltpu.sync_copy(main_ref.at[pl.ds(w * BLKS_PER_TILE, BLKS_PER_TILE)],
                        vref)

    @pl.when(w == NW - 1)
    def _mixed(_=None):
        pltpu.sync_copy(main_ref.at[pl.ds(MAIN_LAST_BASE, LASTW_MAIN)],
                        vref.at[pl.ds(0, LASTW_MAIN)])
        pltpu.sync_copy(pad_ref, vref.at[pl.ds(LASTW_MAIN, PAD_BLKS)])


@functools.cache
def _make_agg(H):
    @functools.partial(
        pl.kernel,
        out_type=jax.ShapeDtypeStruct((NC, NPAD, H), jnp.float32),
        mesh=_mesh(),
        compiler_params=pltpu.CompilerParams(use_tc_tiling_on_sc=False),
        scratch_types=[
            pltpu.VMEM((BLKS_PER_TILE, BLK), jnp.int32),
            pltpu.VMEM((BLKS_PER_TILE, BLK), jnp.int32),
            pltpu.VMEM((RPT, H), jnp.float32),
            pltpu.VMEM_SHARED((NPAD, H), jnp.float32),
        ]
        + [pltpu.VMEM((BLK, H), jnp.float32) for _ in range(NBUF)]
        + [pltpu.SemaphoreType.DMA for _ in range(2 * NBUF)],
    )
    def agg(hs, msrc, mdst, psrc, pdst, out, src_v, dst_v, zer_v, acc_sh, *rb):
        rows = rb[:NBUF]
        gsem = rb[NBUF:2 * NBUF]
        ssem = rb[2 * NBUF:]
        cid = lax.axis_index("c")
        sid = lax.axis_index("s")
        w = cid * NS + sid

        _fill_zeros(zer_v, RPT, H)
        pltpu.sync_copy(zer_v, acc_sh.at[pl.ds(sid * RPT, RPT)])
        plsc.subcore_barrier()

        _load_indices(msrc, psrc, src_v, w)
        _load_indices(mdst, pdst, dst_v, w)

        for b in range(NBUF):
            pltpu.async_copy(hs.at[src_v.at[b]], rows[b], gsem[b])

        @pl.loop(0, BLKS_PER_TILE, step=NBUF)
        def _chunk(j):
            for b in range(NBUF):
                pltpu.make_async_copy(hs.at[pl.ds(0, BLK)], rows[b],
                                      gsem[b]).wait()
                pltpu.async_copy(rows[b], acc_sh.at[dst_v.at[j + b]],
                                 ssem[b], add=True)
            for b in range(NBUF):
                pltpu.make_async_copy(rows[b], acc_sh.at[dst_v.at[j]],
                                      ssem[b]).wait()

                @pl.when(j + b + NBUF < BLKS_PER_TILE)
                def _refill(b=b):
                    pltpu.async_copy(hs.at[src_v.at[j + b + NBUF]],
                                     rows[b], gsem[b])

        plsc.subcore_barrier()
        pltpu.sync_copy(acc_sh.at[pl.ds(sid * RPT, RPT)],
                        out.at[cid, pl.ds(sid * RPT, RPT)])

    return agg


DEGL = 16

F16 = N * H1 // 128
F16P = NPAD * H1 // 128
F32 = N * H2 // 128
F32P = NPAD * H2 // 128


@functools.cache
def _make_deg():
    @functools.partial(
        pl.kernel,
        out_type=jax.ShapeDtypeStruct((NC, NPAD, DEGL), jnp.float32),
        mesh=_mesh(),
        compiler_params=pltpu.CompilerParams(use_tc_tiling_on_sc=False),
        scratch_types=[
            pltpu.VMEM((BLKS_PER_TILE, BLK), jnp.int32),
            pltpu.VMEM((BLK, DEGL), jnp.float32),
            pltpu.VMEM_SHARED((NPAD, DEGL), jnp.float32),
            pltpu.SemaphoreType.DMA,
        ],
    )
    def _deg(mdst, pdst, ones8, zer8, out, dst_v, ones_v, acc_sh, dsem):
        cid = lax.axis_index("c")
        sid = lax.axis_index("s")
        w = cid * NS + sid

        pltpu.sync_copy(ones8, ones_v)
        pltpu.sync_copy(zer8, acc_sh.at[pl.ds(sid * RPT, RPT)])
        plsc.subcore_barrier()

        _load_indices(mdst, pdst, dst_v, w)

        @pl.loop(0, BLKS_PER_TILE)
        def _edge_block(j):
            pltpu.async_copy(ones_v, acc_sh.at[dst_v.at[j]], dsem, add=True)

        @pl.loop(0, BLKS_PER_TILE)
        def _drain(j):
            pltpu.make_async_copy(ones_v, acc_sh.at[dst_v.at[0]], dsem).wait()

        plsc.subcore_barrier()
        pltpu.sync_copy(acc_sh.at[pl.ds(sid * RPT, RPT)],
                        out.at[cid, pl.ds(sid * RPT, RPT)])

    return _deg


def _tc01_body(degf_ref, xf_ref, bdw1_ref, h1sf_ref, ds16_ref):
    d = degf_ref[0, :F16, :] + degf_ref[1, :F16, :] + 1.0
    ds = lax.rsqrt(d)
    h1 = jnp.dot(xf_ref[...], bdw1_ref[...],
                 preferred_element_type=jnp.float32)
    h1sf_ref[...] = h1 * ds
    ds16_ref[...] = ds


def _tc2_body(p_ref, h1sf_ref, ds16_ref, b1f_ref, bdw2_ref, rsel_ref,
              h2sf_ref, ds32_ref):
    agg = p_ref[0, :F16, :] + p_ref[1, :F16, :] + h1sf_ref[...]
    z1 = jnp.maximum(agg * ds16_ref[...] + b1f_ref[...], 0.0)
    h2sf_ref[...] = jnp.dot(z1 * ds16_ref[...], bdw2_ref[...],
                            preferred_element_type=jnp.float32)
    ds32_ref[...] = jnp.dot(ds16_ref[...], rsel_ref[...],
                            preferred_element_type=jnp.float32)


def _tc3_body(q_ref, h2sf_ref, ds32_ref, b2f_ref, wfcf_ref, bfc_ref, out_ref):
    agg = q_ref[0, :F32, :] + q_ref[1, :F32, :] + h2sf_ref[...]
    z2 = jnp.maximum(agg * ds32_ref[...] + b2f_ref[...], 0.0)
    out_ref[...] = jnp.sum(z2 * wfcf_ref[...]) * (1.0 / N) + bfc_ref[...]


def kernel(x, edge_index, W1, b1, W2, b2, Wfc, bfc):
    ei = edge_index.astype(jnp.int32)
    msrc = ei[0].reshape(MAIN_BLKS, BLK)
    mdst = ei[1].reshape(MAIN_BLKS, BLK)
    ar = jnp.arange(PAD_BLKS * BLK, dtype=jnp.int32)
    psrc = ((ar * 97) % N).reshape(PAD_BLKS, BLK)
    pdst = (N + ar % (NPAD - N)).reshape(PAD_BLKS, BLK)

    ones16 = jnp.ones((BLK, DEGL), jnp.float32)
    zer16 = jnp.zeros((RPT, DEGL), jnp.float32)
    degp = _make_deg()(mdst, pdst, ones16, zer16)
    degf = degp.reshape(NC, F16P, 128)

    eye8 = jnp.eye(8, dtype=jnp.float32)
    bdw1 = jnp.kron(eye8, W1)
    bdw2 = jnp.kron(eye8, W2)
    b1f = jnp.tile(b1, 8)
    b2f = jnp.tile(b2, 4)
    wfcf = jnp.tile(Wfc[:, 0], 4)
    xf = x.reshape(N * D_FEAT // 1024, 1024)

    h1sf, ds16 = pl.pallas_call(
        _tc01_body,
        out_shape=[
            jax.ShapeDtypeStruct((F16, 128), jnp.float32),
            jax.ShapeDtypeStruct((F16, 128), jnp.float32),
        ],
    )(degf, xf, bdw1)

    p1 = _make_agg(H1)(h1sf.reshape(N, H1), msrc, mdst, psrc, pdst)

    rsel = jnp.kron(eye8, jnp.full((16, 32), 1.0 / 16, jnp.float32))

    h2sf256, ds32_256 = pl.pallas_call(
        _tc2_body,
        out_shape=[
            jax.ShapeDtypeStruct((F16, 256), jnp.float32),
            jax.ShapeDtypeStruct((F16, 256), jnp.float32),
        ],
    )(p1.reshape(NC, F16P, 128), h1sf, ds16, b1f, bdw2, rsel)
    h2sf = h2sf256.reshape(F32, 128)
    ds32 = ds32_256.reshape(F32, 128)

    p2 = _make_agg(H2)(h2sf.reshape(N, H2), msrc, mdst, psrc, pdst)

    out = pl.pallas_call(
        _tc3_body,
        out_shape=jax.ShapeDtypeStruct((1,), jnp.float32),
    )(p2.reshape(NC, F32P, 128), h2sf, ds32, b2f, wfcf, bfc)

    return out
